# baseline (speedup 1.0000x reference)
import functools

import jax
import jax.numpy as jnp
from jax import lax
from jax.experimental import pallas as pl
from jax.experimental.pallas import tpu as pltpu
from jax.experimental.pallas import tpu_sc as plsc

N, E, D, DE, H = 10000, 320000, 128, 16, 64
EPAD = 331776
BE = 4096
BN = 1000
F = 64

NT = 16
CHUNK = 128
PT1 = EPAD // NT
NC1 = PT1 // CHUNK
PT2 = EPAD // (2 * NT)
NC2 = PT2 // CHUNK
NP = 10240
NSL = NP // NT
GB = 3


def _sc_mesh():
    return plsc.VectorSubcoreMesh(core_axis_name="c", subcore_axis_name="s")


def _leaky(v):
    return jnp.where(v >= 0, v, 0.01 * v)


def _emlp_body(ea_ref, W1c_ref, b1c_ref, W2c_ref, b2c_ref, w1_ref, w2_ref):
    i = pl.program_id(0)
    h = jnp.maximum(ea_ref[...] @ W1c_ref[...] + b1c_ref[...][None, :], 0.0)
    z = h @ W2c_ref[...] + b2c_ref[...][None, :]
    w12 = jax.nn.sigmoid(z)
    gid = i * BE + lax.broadcasted_iota(jnp.int32, (BE,), 0)
    valid = gid < E
    w1_ref[...] = jnp.where(valid, w12[:, 0], 0.0)
    w2_ref[...] = jnp.where(valid, w12[:, 1], 0.0)


def _edge_weights(ea_p, mW1_1, mb1_1, mW2_1, mb2_1, mW1_2, mb1_2, mW2_2, mb2_2):
    W1c = jnp.concatenate([mW1_1, mW1_2], axis=1)
    b1c = jnp.concatenate([mb1_1, mb1_2])
    W2c = jnp.zeros((2 * H, 8), jnp.float32)
    W2c = W2c.at[:H, 0].set(mW2_1[:, 0]).at[H:, 1].set(mW2_2[:, 0])
    b2c = jnp.zeros((8,), jnp.float32)
    b2c = b2c.at[0].set(mb2_1[0]).at[1].set(mb2_2[0])
    return pl.pallas_call(
        _emlp_body,
        grid=(EPAD // BE,),
        in_specs=[
            pl.BlockSpec((BE, DE), lambda i: (i, 0)),
            pl.BlockSpec((DE, 2 * H), lambda i: (0, 0)),
            pl.BlockSpec((2 * H,), lambda i: (0,)),
            pl.BlockSpec((2 * H, 8), lambda i: (0, 0)),
            pl.BlockSpec((8,), lambda i: (0,)),
        ],
        out_specs=[
            pl.BlockSpec((BE,), lambda i: (i,)),
            pl.BlockSpec((BE,), lambda i: (i,)),
        ],
        out_shape=[
            jax.ShapeDtypeStruct((EPAD,), jnp.float32),
            jax.ShapeDtypeStruct((EPAD,), jnp.float32),
        ],
    )(ea_p, W1c, b1c, W2c, b2c)


def _u_body(x_ref, deg_ref, u_ref):
    dis = lax.rsqrt(deg_ref[...] + 1.0)
    u_ref[0] = dis * x_ref[:, :F]
    u_ref[1] = dis * x_ref[:, F:]


def _u1(x, deg1):
    return pl.pallas_call(
        _u_body,
        grid=(N // BN,),
        in_specs=[
            pl.BlockSpec((BN, D), lambda i: (i, 0)),
            pl.BlockSpec((BN, 1), lambda i: (i, 0)),
        ],
        out_specs=pl.BlockSpec((2, BN, F), lambda i: (0, i, 0)),
        out_shape=jax.ShapeDtypeStruct((2, NP, F), jnp.float32),
    )(x, deg1)


def _x1y2_body(s1_ref, u_ref, deg1_ref, b1_ref, lin1_ref, lin2_ref,
               deg2_ref, y2_ref):
    dis1 = lax.rsqrt(deg1_ref[...] + 1.0)
    t0 = s1_ref[0, 0] + s1_ref[1, 0] - u_ref[0]
    t1 = s1_ref[0, 1] + s1_ref[1, 1] - u_ref[1]
    t = jnp.concatenate([t0, t1], axis=1)
    x1 = _leaky(dis1 * (t @ lin1_ref[...]) + b1_ref[...][None, :])
    xl2 = x1 @ lin2_ref[...]
    dis2 = lax.rsqrt(deg2_ref[...] + 1.0)
    y2_ref[0] = dis2 * xl2[:, :F]
    y2_ref[1] = dis2 * xl2[:, F:]


def _x1y2(s1, u, deg1, b1, lin1, lin2, deg2):
    return pl.pallas_call(
        _x1y2_body,
        grid=(N // BN,),
        in_specs=[
            pl.BlockSpec((2, 2, BN, F), lambda i: (0, 0, i, 0)),
            pl.BlockSpec((2, BN, F), lambda i: (0, i, 0)),
            pl.BlockSpec((BN, 1), lambda i: (i, 0)),
            pl.BlockSpec((2 * D,), lambda i: (0,)),
            pl.BlockSpec((D, 2 * D), lambda i: (0, 0)),
            pl.BlockSpec((2 * D, D), lambda i: (0, 0)),
            pl.BlockSpec((BN, 1), lambda i: (i, 0)),
        ],
        out_specs=pl.BlockSpec((2, BN, F), lambda i: (0, i, 0)),
        out_shape=jax.ShapeDtypeStruct((2, NP, F), jnp.float32),
    )(s1, u, deg1, b1, lin1, lin2, deg2)


def _final_body(s2_ref, y2_ref, x_ref, deg2_ref, b2_ref, fcW_ref,
                fcb_ref, out_ref):
    i = pl.program_id(0)
    dis2 = lax.rsqrt(deg2_ref[...] + 1.0)
    ms = []
    for h in range(2):
        x2h = (dis2 * (s2_ref[0, h] + s2_ref[1, h] - y2_ref[h])
               + b2_ref[h][None, :] + x_ref[:, h * F:(h + 1) * F])
        ms.append(jnp.max(x2h, axis=0, keepdims=True))
    m = jnp.concatenate(ms, axis=1)

    @pl.when(i == 0)
    def _():
        out_ref[...] = m

    @pl.when(i > 0)
    def _():
        out_ref[...] = jnp.maximum(out_ref[...], m)

    @pl.when(i == pl.num_programs(0) - 1)
    def _():
        g = _leaky(out_ref[...])
        out_ref[...] = g @ fcW_ref[...] + fcb_ref[...][None, :]


def _final(s2, y2, x, deg2, b2, fcW, fcb):
    b2r = b2.reshape(2, F)
    return pl.pallas_call(
        _final_body,
        grid=(N // BN,),
        in_specs=[
            pl.BlockSpec((2, 2, BN, F), lambda i: (0, 0, i, 0)),
            pl.BlockSpec((2, BN, F), lambda i: (0, i, 0)),
            pl.BlockSpec((BN, D), lambda i: (i, 0)),
            pl.BlockSpec((BN, 1), lambda i: (i, 0)),
            pl.BlockSpec((2, F), lambda i: (0, 0)),
            pl.BlockSpec((D, D), lambda i: (0, 0)),
            pl.BlockSpec((D,), lambda i: (0,)),
        ],
        out_specs=pl.BlockSpec((1, D), lambda i: (0, 0)),
        out_shape=jax.ShapeDtypeStruct((1, D), jnp.float32),
    )(s2, y2, x, deg2, b2r, fcW, fcb)


def _sc_deg(col2_p, w12_p, w22_p):
    @functools.partial(
        pl.kernel,
        out_type=[jax.ShapeDtypeStruct((NP,), jnp.float32),
                  jax.ShapeDtypeStruct((NP,), jnp.float32)],
        mesh=_sc_mesh(),
        compiler_params=pltpu.CompilerParams(use_tc_tiling_on_sc=False),
        scratch_types=[
            pltpu.VMEM((NC1, CHUNK), jnp.int32),
            pltpu.VMEM((NC1, CHUNK), jnp.float32),
            pltpu.VMEM((NP,), jnp.float32),
            pltpu.VMEM_SHARED((NP,), jnp.float32),
        ],
    )
    def deg_kernel(col_hbm, w1_hbm, w2_hbm, d1_hbm, d2_hbm,
                   col2d, w2d, z_v, acc):
        c = lax.axis_index("c")
        s = lax.axis_index("s")
        pltpu.sync_copy(col_hbm.at[pl.ds(s * NC1, NC1)], col2d)

        @pl.when(c == 0)
        def _():
            pltpu.sync_copy(w1_hbm.at[pl.ds(s * NC1, NC1)], w2d)

        @pl.when(c == 1)
        def _():
            pltpu.sync_copy(w2_hbm.at[pl.ds(s * NC1, NC1)], w2d)

        @pl.when(s == 0)
        def _():
            def zb(i, carry):
                z_v[pl.ds(i * 16, 16)] = jnp.zeros((16,), jnp.float32)
                return carry
            lax.fori_loop(0, NP // 16, zb, 0)
            pltpu.sync_copy(z_v, acc)

        plsc.subcore_barrier()

        def body(k, carry):
            pltpu.sync_copy(w2d.at[k], acc.at[col2d.at[k]], add=True)
            return carry
        lax.fori_loop(0, NC1, body, 0)

        plsc.subcore_barrier()

        @pl.when(s == 0)
        def _():
            @pl.when(c == 0)
            def _():
                pltpu.sync_copy(acc, d1_hbm)

            @pl.when(c == 1)
            def _():
                pltpu.sync_copy(acc, d2_hbm)

    return deg_kernel(col2_p, w12_p, w22_p)


def _agg_pass(y_grp, s_grp, row2d, col2d, w_all, bufs, gsems, ssems, stage,
              acc, tile, n_chunks):
    nb = tile * NSL
    for i in range(NSL // CHUNK):
        sl = pl.ds(nb + i * CHUNK, CHUNK)
        pltpu.sync_copy(y_grp.at[sl], stage)
        pltpu.sync_copy(stage, acc.at[sl])
    plsc.subcore_barrier()

    for b in range(GB):
        pltpu.async_copy(y_grp.at[row2d.at[b]], bufs[b], gsems[b])

    def outer(k0, carry):
        for b in range(GB):
            k = k0 * GB + b
            bnxt = (b + 2) % GB

            @pl.when(jnp.logical_and(k >= 1, k + 2 < n_chunks))
            def _():
                pltpu.make_async_copy(bufs[bnxt], acc.at[col2d.at[0]],
                                      ssems[bnxt]).wait()
                pltpu.async_copy(y_grp.at[row2d.at[k + 2]],
                                 bufs[bnxt], gsems[bnxt])

            pltpu.make_async_copy(y_grp.at[row2d.at[0]],
                                  bufs[b], gsems[b]).wait()
            kb = k * CHUNK

            def scale(g, carry2):
                e0 = g * 16
                w16 = w_all[pl.ds(kb + e0, 16)]
                for j in range(16):
                    ws = jnp.broadcast_to(w16[j], (16,))
                    for f in range(F // 16):
                        sl = pl.ds(f * 16, 16)
                        bufs[b][e0 + j, sl] = bufs[b][e0 + j, sl] * ws
                return carry2
            lax.fori_loop(0, CHUNK // 16, scale, 0)

            pltpu.async_copy(bufs[b], acc.at[col2d.at[k]], ssems[b], add=True)
        return carry
    lax.fori_loop(0, n_chunks // GB, outer, 0)

    for b in range(GB):
        pltpu.make_async_copy(bufs[b], acc.at[col2d.at[0]], ssems[b]).wait()

    plsc.subcore_barrier()
    for i in range(NSL // CHUNK):
        sl = pl.ds(nb + i * CHUNK, CHUNK)
        pltpu.sync_copy(acc.at[sl], stage)
        pltpu.sync_copy(stage, s_grp.at[sl])


def _make_agg(ngrp):
    half = EPAD // (2 * CHUNK)

    @functools.partial(
        pl.kernel,
        out_type=jax.ShapeDtypeStruct((2, ngrp, NP, F), jnp.float32),
        mesh=_sc_mesh(),
        compiler_params=pltpu.CompilerParams(use_tc_tiling_on_sc=False),
        scratch_types=[
            pltpu.VMEM((NC2, CHUNK), jnp.int32),
            pltpu.VMEM((NC2, CHUNK), jnp.int32),
            pltpu.VMEM((PT2,), jnp.float32),
            pltpu.VMEM((CHUNK, F), jnp.float32),
            pltpu.VMEM((CHUNK, F), jnp.float32),
            pltpu.VMEM((CHUNK, F), jnp.float32),
            pltpu.VMEM((CHUNK, F), jnp.float32),
            pltpu.VMEM_SHARED((NP, F), jnp.float32),
            pltpu.SemaphoreType.DMA,
            pltpu.SemaphoreType.DMA,
            pltpu.SemaphoreType.DMA,
            pltpu.SemaphoreType.DMA,
            pltpu.SemaphoreType.DMA,
            pltpu.SemaphoreType.DMA,
        ],
    )
    def agg(row_hbm, col_hbm, w_hbm, y_hbm, s_hbm,
            row2d, col2d, w_all, buf0, buf1, buf2, stage, acc,
            sem0, sem1, sem2, sem3, sem4, sem5):
        c = lax.axis_index("c")
        s = lax.axis_index("s")
        cb = c * half + s * NC2
        pltpu.sync_copy(row_hbm.at[pl.ds(cb, NC2)], row2d)
        pltpu.sync_copy(col_hbm.at[pl.ds(cb, NC2)], col2d)
        pltpu.sync_copy(w_hbm.at[pl.ds(c * (EPAD // 2) + s * PT2, PT2)], w_all)
        for g in range(ngrp):
            plsc.subcore_barrier()
            _agg_pass(y_hbm.at[g], s_hbm.at[c].at[g], row2d, col2d, w_all,
                      (buf0, buf1, buf2), (sem0, sem1, sem2),
                      (sem3, sem4, sem5), stage, acc, s, NC2)

    return agg


def kernel(x, edge_index, edge_attr, batch, lin1, mW1_1, mb1_1, mW2_1, mb2_1,
           b1, lin2, mW1_2, mb1_2, mW2_2, mb2_2, b2, fcW, fcb):
    pad = EPAD - E
    ea_p = jnp.pad(edge_attr, ((0, pad), (0, 0)))
    row_p = jnp.pad(edge_index[0], (0, pad))
    col_p = jnp.pad(edge_index[1], (0, pad))
    row2_p = row_p.reshape(EPAD // CHUNK, CHUNK)
    col2_p = col_p.reshape(EPAD // CHUNK, CHUNK)

    w1_p, w2_p = _edge_weights(ea_p, mW1_1, mb1_1, mW2_1, mb2_1,
                               mW1_2, mb1_2, mW2_2, mb2_2)

    w12_p = w1_p.reshape(EPAD // CHUNK, CHUNK)
    w22_p = w2_p.reshape(EPAD // CHUNK, CHUNK)
    d1, d2 = _sc_deg(col2_p, w12_p, w22_p)
    deg1, deg2 = d1[:, None], d2[:, None]

    u = _u1(x, deg1)
    s1 = _make_agg(2)(row2_p, col2_p, w1_p, u)
    y2 = _x1y2(s1, u, deg1, b1, lin1, lin2, deg2)
    s2 = _make_agg(2)(row2_p, col2_p, w2_p, y2)

    return _final(s2, y2, x, deg2, b2, fcW, fcb)


# speedup vs baseline: 1.1424x; 1.1424x over previous
import functools

import jax
import jax.numpy as jnp
from jax import lax
from jax.experimental import pallas as pl
from jax.experimental.pallas import tpu as pltpu
from jax.experimental.pallas import tpu_sc as plsc

N, E, D, DE, H = 10000, 320000, 128, 16, 64
EPAD = 331776
BE = 4096
BN = 1000
F = 64

NT = 16
CHUNK = 128
PT1 = EPAD // NT
NC1 = PT1 // CHUNK
PT2 = EPAD // (2 * NT)
NC2 = PT2 // CHUNK
NP = 10240
NSL = NP // NT
GB = 3


def _sc_mesh():
    return plsc.VectorSubcoreMesh(core_axis_name="c", subcore_axis_name="s")


def _leaky(v):
    return jnp.where(v >= 0, v, 0.01 * v)


def _emlp_body(ea_ref, W1c_ref, b1c_ref, W2c_ref, b2c_ref, w1_ref, w2_ref):
    i = pl.program_id(0)
    h = jnp.maximum(ea_ref[...] @ W1c_ref[...] + b1c_ref[...][None, :], 0.0)
    z = h @ W2c_ref[...] + b2c_ref[...][None, :]
    w12 = jax.nn.sigmoid(z)
    gid = i * BE + lax.broadcasted_iota(jnp.int32, (BE,), 0)
    valid = gid < E
    w1_ref[...] = jnp.where(valid, w12[:, 0], 0.0)
    w2_ref[...] = jnp.where(valid, w12[:, 1], 0.0)


def _edge_weights(ea_p, mW1_1, mb1_1, mW2_1, mb2_1, mW1_2, mb1_2, mW2_2, mb2_2):
    W1c = jnp.concatenate([mW1_1, mW1_2], axis=1)
    b1c = jnp.concatenate([mb1_1, mb1_2])
    W2c = jnp.zeros((2 * H, 8), jnp.float32)
    W2c = W2c.at[:H, 0].set(mW2_1[:, 0]).at[H:, 1].set(mW2_2[:, 0])
    b2c = jnp.zeros((8,), jnp.float32)
    b2c = b2c.at[0].set(mb2_1[0]).at[1].set(mb2_2[0])
    return pl.pallas_call(
        _emlp_body,
        grid=(EPAD // BE,),
        in_specs=[
            pl.BlockSpec((BE, DE), lambda i: (i, 0)),
            pl.BlockSpec((DE, 2 * H), lambda i: (0, 0)),
            pl.BlockSpec((2 * H,), lambda i: (0,)),
            pl.BlockSpec((2 * H, 8), lambda i: (0, 0)),
            pl.BlockSpec((8,), lambda i: (0,)),
        ],
        out_specs=[
            pl.BlockSpec((BE,), lambda i: (i,)),
            pl.BlockSpec((BE,), lambda i: (i,)),
        ],
        out_shape=[
            jax.ShapeDtypeStruct((EPAD,), jnp.float32),
            jax.ShapeDtypeStruct((EPAD,), jnp.float32),
        ],
    )(ea_p, W1c, b1c, W2c, b2c)


def _u_body(x_ref, deg_ref, u_ref, ub_ref):
    dis = lax.rsqrt(deg_ref[...] + 1.0)
    u0 = dis * x_ref[:, :F]
    u1 = dis * x_ref[:, F:]
    u_ref[0] = u0
    u_ref[1] = u1
    ub_ref[0] = u0.astype(jnp.bfloat16)
    ub_ref[1] = u1.astype(jnp.bfloat16)


def _u1(x, deg1):
    return pl.pallas_call(
        _u_body,
        grid=(N // BN,),
        in_specs=[
            pl.BlockSpec((BN, D), lambda i: (i, 0)),
            pl.BlockSpec((BN, 1), lambda i: (i, 0)),
        ],
        out_specs=[
            pl.BlockSpec((2, BN, F), lambda i: (0, i, 0)),
            pl.BlockSpec((2, BN, F), lambda i: (0, i, 0)),
        ],
        out_shape=[
            jax.ShapeDtypeStruct((2, NP, F), jnp.float32),
            jax.ShapeDtypeStruct((2, NP, F), jnp.bfloat16),
        ],
    )(x, deg1)


def _x1y2_body(s1_ref, u_ref, deg1_ref, b1_ref, lin1_ref, lin2_ref,
               deg2_ref, y2_ref, y2b_ref):
    dis1 = lax.rsqrt(deg1_ref[...] + 1.0)
    t0 = s1_ref[0, 0] + s1_ref[1, 0] - u_ref[0]
    t1 = s1_ref[0, 1] + s1_ref[1, 1] - u_ref[1]
    t = jnp.concatenate([t0, t1], axis=1)
    x1 = _leaky(dis1 * (t @ lin1_ref[...]) + b1_ref[...][None, :])
    xl2 = x1 @ lin2_ref[...]
    dis2 = lax.rsqrt(deg2_ref[...] + 1.0)
    y20 = dis2 * xl2[:, :F]
    y21 = dis2 * xl2[:, F:]
    y2_ref[0] = y20
    y2_ref[1] = y21
    y2b_ref[0] = y20.astype(jnp.bfloat16)
    y2b_ref[1] = y21.astype(jnp.bfloat16)


def _x1y2(s1, u, deg1, b1, lin1, lin2, deg2):
    return pl.pallas_call(
        _x1y2_body,
        grid=(N // BN,),
        in_specs=[
            pl.BlockSpec((2, 2, BN, F), lambda i: (0, 0, i, 0)),
            pl.BlockSpec((2, BN, F), lambda i: (0, i, 0)),
            pl.BlockSpec((BN, 1), lambda i: (i, 0)),
            pl.BlockSpec((2 * D,), lambda i: (0,)),
            pl.BlockSpec((D, 2 * D), lambda i: (0, 0)),
            pl.BlockSpec((2 * D, D), lambda i: (0, 0)),
            pl.BlockSpec((BN, 1), lambda i: (i, 0)),
        ],
        out_specs=[
            pl.BlockSpec((2, BN, F), lambda i: (0, i, 0)),
            pl.BlockSpec((2, BN, F), lambda i: (0, i, 0)),
        ],
        out_shape=[
            jax.ShapeDtypeStruct((2, NP, F), jnp.float32),
            jax.ShapeDtypeStruct((2, NP, F), jnp.bfloat16),
        ],
    )(s1, u, deg1, b1, lin1, lin2, deg2)


def _final_body(s2_ref, y2_ref, x_ref, deg2_ref, b2_ref, fcW_ref,
                fcb_ref, out_ref):
    i = pl.program_id(0)
    dis2 = lax.rsqrt(deg2_ref[...] + 1.0)
    ms = []
    for h in range(2):
        x2h = (dis2 * (s2_ref[0, h] + s2_ref[1, h] - y2_ref[h])
               + b2_ref[h][None, :] + x_ref[:, h * F:(h + 1) * F])
        ms.append(jnp.max(x2h, axis=0, keepdims=True))
    m = jnp.concatenate(ms, axis=1)

    @pl.when(i == 0)
    def _():
        out_ref[...] = m

    @pl.when(i > 0)
    def _():
        out_ref[...] = jnp.maximum(out_ref[...], m)

    @pl.when(i == pl.num_programs(0) - 1)
    def _():
        g = _leaky(out_ref[...])
        out_ref[...] = g @ fcW_ref[...] + fcb_ref[...][None, :]


def _final(s2, y2, x, deg2, b2, fcW, fcb):
    b2r = b2.reshape(2, F)
    return pl.pallas_call(
        _final_body,
        grid=(N // BN,),
        in_specs=[
            pl.BlockSpec((2, 2, BN, F), lambda i: (0, 0, i, 0)),
            pl.BlockSpec((2, BN, F), lambda i: (0, i, 0)),
            pl.BlockSpec((BN, D), lambda i: (i, 0)),
            pl.BlockSpec((BN, 1), lambda i: (i, 0)),
            pl.BlockSpec((2, F), lambda i: (0, 0)),
            pl.BlockSpec((D, D), lambda i: (0, 0)),
            pl.BlockSpec((D,), lambda i: (0,)),
        ],
        out_specs=pl.BlockSpec((1, D), lambda i: (0, 0)),
        out_shape=jax.ShapeDtypeStruct((1, D), jnp.float32),
    )(s2, y2, x, deg2, b2r, fcW, fcb)


def _sc_deg(col2_p, w12_p, w22_p):
    @functools.partial(
        pl.kernel,
        out_type=[jax.ShapeDtypeStruct((NP,), jnp.float32),
                  jax.ShapeDtypeStruct((NP,), jnp.float32)],
        mesh=_sc_mesh(),
        compiler_params=pltpu.CompilerParams(use_tc_tiling_on_sc=False),
        scratch_types=[
            pltpu.VMEM((NC1, CHUNK), jnp.int32),
            pltpu.VMEM((NC1, CHUNK), jnp.float32),
            pltpu.VMEM((NP,), jnp.float32),
            pltpu.VMEM_SHARED((NP,), jnp.float32),
        ],
    )
    def deg_kernel(col_hbm, w1_hbm, w2_hbm, d1_hbm, d2_hbm,
                   col2d, w2d, z_v, acc):
        c = lax.axis_index("c")
        s = lax.axis_index("s")
        pltpu.sync_copy(col_hbm.at[pl.ds(s * NC1, NC1)], col2d)

        @pl.when(c == 0)
        def _():
            pltpu.sync_copy(w1_hbm.at[pl.ds(s * NC1, NC1)], w2d)

        @pl.when(c == 1)
        def _():
            pltpu.sync_copy(w2_hbm.at[pl.ds(s * NC1, NC1)], w2d)

        @pl.when(s == 0)
        def _():
            def zb(i, carry):
                z_v[pl.ds(i * 16, 16)] = jnp.zeros((16,), jnp.float32)
                return carry
            lax.fori_loop(0, NP // 16, zb, 0)
            pltpu.sync_copy(z_v, acc)

        plsc.subcore_barrier()

        def body(k, carry):
            pltpu.sync_copy(w2d.at[k], acc.at[col2d.at[k]], add=True)
            return carry
        lax.fori_loop(0, NC1, body, 0)

        plsc.subcore_barrier()

        @pl.when(s == 0)
        def _():
            @pl.when(c == 0)
            def _():
                pltpu.sync_copy(acc, d1_hbm)

            @pl.when(c == 1)
            def _():
                pltpu.sync_copy(acc, d2_hbm)

    return deg_kernel(col2_p, w12_p, w22_p)


def _agg_pass(y_grp, yb_grp, s_grp, row2d, col2d, w_all, gbufs, sbufs, gsems,
              ssems, stage, acc, tile, n_chunks):
    nb = tile * NSL
    for i in range(NSL // CHUNK):
        sl = pl.ds(nb + i * CHUNK, CHUNK)
        pltpu.sync_copy(y_grp.at[sl], stage)
        pltpu.sync_copy(stage, acc.at[sl])
    plsc.subcore_barrier()

    for b in range(GB):
        pltpu.async_copy(yb_grp.at[row2d.at[b]], gbufs[b], gsems[b])

    def outer(k0, carry):
        for b in range(GB):
            k = k0 * GB + b
            bnxt = (b + 2) % GB

            @pl.when(jnp.logical_and(k >= 1, k + 2 < n_chunks))
            def _():
                pltpu.async_copy(yb_grp.at[row2d.at[k + 2]],
                                 gbufs[bnxt], gsems[bnxt])

            pltpu.make_async_copy(yb_grp.at[row2d.at[0]],
                                  gbufs[b], gsems[b]).wait()

            @pl.when(k >= GB)
            def _():
                pltpu.make_async_copy(sbufs[b], acc.at[col2d.at[0]],
                                      ssems[b]).wait()
            kb = k * CHUNK

            def scale(g, carry2):
                e0 = g * 16
                w16 = w_all[pl.ds(kb + e0, 16)]
                for j in range(16):
                    ws = jnp.broadcast_to(w16[j], (16,))
                    for f in range(F // 16):
                        sl = pl.ds(f * 16, 16)
                        v = gbufs[b][e0 + j, sl].astype(jnp.float32)
                        sbufs[b][e0 + j, sl] = v * ws
                return carry2
            lax.fori_loop(0, CHUNK // 16, scale, 0)

            pltpu.async_copy(sbufs[b], acc.at[col2d.at[k]], ssems[b], add=True)
        return carry
    lax.fori_loop(0, n_chunks // GB, outer, 0)

    for b in range(GB):
        pltpu.make_async_copy(sbufs[b], acc.at[col2d.at[0]], ssems[b]).wait()

    plsc.subcore_barrier()
    for i in range(NSL // CHUNK):
        sl = pl.ds(nb + i * CHUNK, CHUNK)
        pltpu.sync_copy(acc.at[sl], stage)
        pltpu.sync_copy(stage, s_grp.at[sl])


def _make_agg(ngrp):
    half = EPAD // (2 * CHUNK)

    @functools.partial(
        pl.kernel,
        out_type=jax.ShapeDtypeStruct((2, ngrp, NP, F), jnp.float32),
        mesh=_sc_mesh(),
        compiler_params=pltpu.CompilerParams(use_tc_tiling_on_sc=False),
        scratch_types=[
            pltpu.VMEM((NC2, CHUNK), jnp.int32),
            pltpu.VMEM((NC2, CHUNK), jnp.int32),
            pltpu.VMEM((PT2,), jnp.float32),
            pltpu.VMEM((CHUNK, F), jnp.bfloat16),
            pltpu.VMEM((CHUNK, F), jnp.bfloat16),
            pltpu.VMEM((CHUNK, F), jnp.bfloat16),
            pltpu.VMEM((CHUNK, F), jnp.float32),
            pltpu.VMEM((CHUNK, F), jnp.float32),
            pltpu.VMEM((CHUNK, F), jnp.float32),
            pltpu.VMEM((CHUNK, F), jnp.float32),
            pltpu.VMEM_SHARED((NP, F), jnp.float32),
            pltpu.SemaphoreType.DMA,
            pltpu.SemaphoreType.DMA,
            pltpu.SemaphoreType.DMA,
            pltpu.SemaphoreType.DMA,
            pltpu.SemaphoreType.DMA,
            pltpu.SemaphoreType.DMA,
        ],
    )
    def agg(row_hbm, col_hbm, w_hbm, y_hbm, yb_hbm, s_hbm,
            row2d, col2d, w_all, gbuf0, gbuf1, gbuf2, sbuf0, sbuf1, sbuf2,
            stage, acc, sem0, sem1, sem2, sem3, sem4, sem5):
        c = lax.axis_index("c")
        s = lax.axis_index("s")
        cb = c * half + s * NC2
        pltpu.sync_copy(row_hbm.at[pl.ds(cb, NC2)], row2d)
        pltpu.sync_copy(col_hbm.at[pl.ds(cb, NC2)], col2d)
        pltpu.sync_copy(w_hbm.at[pl.ds(c * (EPAD // 2) + s * PT2, PT2)], w_all)
        for g in range(ngrp):
            plsc.subcore_barrier()
            _agg_pass(y_hbm.at[g], yb_hbm.at[g], s_hbm.at[c].at[g],
                      row2d, col2d, w_all, (gbuf0, gbuf1, gbuf2),
                      (sbuf0, sbuf1, sbuf2), (sem0, sem1, sem2),
                      (sem3, sem4, sem5), stage, acc, s, NC2)

    return agg


def kernel(x, edge_index, edge_attr, batch, lin1, mW1_1, mb1_1, mW2_1, mb2_1,
           b1, lin2, mW1_2, mb1_2, mW2_2, mb2_2, b2, fcW, fcb):
    pad = EPAD - E
    ea_p = jnp.pad(edge_attr, ((0, pad), (0, 0)))
    row_p = jnp.pad(edge_index[0], (0, pad))
    col_p = jnp.pad(edge_index[1], (0, pad))
    row2_p = row_p.reshape(EPAD // CHUNK, CHUNK)
    col2_p = col_p.reshape(EPAD // CHUNK, CHUNK)

    w1_p, w2_p = _edge_weights(ea_p, mW1_1, mb1_1, mW2_1, mb2_1,
                               mW1_2, mb1_2, mW2_2, mb2_2)

    w12_p = w1_p.reshape(EPAD // CHUNK, CHUNK)
    w22_p = w2_p.reshape(EPAD // CHUNK, CHUNK)
    d1, d2 = _sc_deg(col2_p, w12_p, w22_p)
    deg1, deg2 = d1[:, None], d2[:, None]

    u, ub = _u1(x, deg1)
    s1 = _make_agg(2)(row2_p, col2_p, w1_p, u, ub)
    y2, y2b = _x1y2(s1, u, deg1, b1, lin1, lin2, deg2)
    s2 = _make_agg(2)(row2_p, col2_p, w2_p, y2, y2b)

    return _final(s2, y2, x, deg2, b2, fcW, fcb)
